# Initial kernel scaffold; baseline (speedup 1.0000x reference)
import jax
import jax.numpy as jnp
from jax.experimental import pallas as pl


def kernel(input, adj, weight):
    raise NotImplementedError("write your pallas kernel here")


# speedup vs baseline: 2.9030x; 2.9030x over previous
import jax
import jax.numpy as jnp
from jax.experimental import pallas as pl


def _support_kernel(x_ref, w_ref, out_ref):
    k = x_ref.shape[1]
    f_out = w_ref.shape[1]
    w = w_ref[...]
    for i in range(k):
        out_ref[:, i * f_out:(i + 1) * f_out] = jnp.dot(
            x_ref[:, i, :], w, preferred_element_type=jnp.float32)


def _spmm_kernel(adj_ref, s_ref, out_ref):
    out_ref[...] = jnp.maximum(
        jnp.dot(adj_ref[...], s_ref[...], preferred_element_type=jnp.float32),
        0.0)


def kernel(input, adj, weight):
    n, k, f_in = input.shape
    f_out = weight.shape[1]

    bn1 = 2000
    s = pl.pallas_call(
        _support_kernel,
        grid=(n // bn1,),
        in_specs=[
            pl.BlockSpec((bn1, k, f_in), lambda i: (i, 0, 0)),
            pl.BlockSpec((f_in, f_out), lambda i: (0, 0)),
        ],
        out_specs=pl.BlockSpec((bn1, k * f_out), lambda i: (i, 0)),
        out_shape=jax.ShapeDtypeStruct((n, k * f_out), jnp.float32),
    )(input, weight)

    bm = 200
    out2d = pl.pallas_call(
        _spmm_kernel,
        grid=(n // bm,),
        in_specs=[
            pl.BlockSpec((bm, n), lambda i: (i, 0)),
            pl.BlockSpec((n, k * f_out), lambda i: (0, 0)),
        ],
        out_specs=pl.BlockSpec((bm, k * f_out), lambda i: (i, 0)),
        out_shape=jax.ShapeDtypeStruct((n, k * f_out), jnp.float32),
    )(adj, s)
    return out2d.reshape(n, k, f_out)


# speedup vs baseline: 3.1343x; 1.0797x over previous
import jax
import jax.numpy as jnp
from jax.experimental import pallas as pl


def _support_kernel(x_ref, w_ref, out_ref):
    k = x_ref.shape[1]
    f_out = w_ref.shape[1]
    w = w_ref[...]
    for i in range(k):
        out_ref[:, i * f_out:(i + 1) * f_out] = jnp.dot(
            x_ref[:, i, :], w, preferred_element_type=jnp.float32)


def _spmm_kernel(adj_ref, s_ref, out_ref):
    out_ref[...] = jnp.maximum(
        jnp.dot(adj_ref[...], s_ref[...], preferred_element_type=jnp.float32),
        0.0)


def kernel(input, adj, weight):
    n, k, f_in = input.shape
    f_out = weight.shape[1]

    bn1 = 2000
    s = pl.pallas_call(
        _support_kernel,
        grid=(n // bn1,),
        in_specs=[
            pl.BlockSpec((bn1, k, f_in), lambda i: (i, 0, 0)),
            pl.BlockSpec((f_in, f_out), lambda i: (0, 0)),
        ],
        out_specs=pl.BlockSpec((bn1, k * f_out), lambda i: (i, 0)),
        out_shape=jax.ShapeDtypeStruct((n, k * f_out), jnp.float32),
    )(input, weight)

    bm = 400
    out2d = pl.pallas_call(
        _spmm_kernel,
        grid=(n // bm,),
        in_specs=[
            pl.BlockSpec((bm, n), lambda i: (i, 0)),
            pl.BlockSpec((n, k * f_out), lambda i: (0, 0)),
        ],
        out_specs=pl.BlockSpec((bm, k * f_out), lambda i: (i, 0)),
        out_shape=jax.ShapeDtypeStruct((n, k * f_out), jnp.float32),
    )(adj, s)
    return out2d.reshape(n, k, f_out)


# speedup vs baseline: 3.1668x; 1.0104x over previous
import jax
import jax.numpy as jnp
from jax.experimental import pallas as pl
from jax.experimental.pallas import tpu as pltpu


def _support_kernel(x_ref, w_ref, out_ref):
    k = x_ref.shape[1]
    f_out = w_ref.shape[1]
    w = w_ref[...]
    for i in range(k):
        out_ref[:, i * f_out:(i + 1) * f_out] = jnp.dot(
            x_ref[:, i, :], w, preferred_element_type=jnp.float32)


def _spmm_kernel(adj_ref, s_ref, out_ref):
    out_ref[...] = jnp.maximum(
        jnp.dot(adj_ref[...], s_ref[...], preferred_element_type=jnp.float32),
        0.0)


def kernel(input, adj, weight):
    n, k, f_in = input.shape
    f_out = weight.shape[1]

    bn1 = 2000
    s = pl.pallas_call(
        _support_kernel,
        grid=(n // bn1,),
        in_specs=[
            pl.BlockSpec((bn1, k, f_in), lambda i: (i, 0, 0)),
            pl.BlockSpec((f_in, f_out), lambda i: (0, 0)),
        ],
        out_specs=pl.BlockSpec((bn1, k * f_out), lambda i: (i, 0)),
        out_shape=jax.ShapeDtypeStruct((n, k * f_out), jnp.float32),
        compiler_params=pltpu.CompilerParams(
            dimension_semantics=("parallel",)),
    )(input, weight)

    bm = 400
    out2d = pl.pallas_call(
        _spmm_kernel,
        grid=(n // bm,),
        in_specs=[
            pl.BlockSpec((bm, n), lambda i: (i, 0)),
            pl.BlockSpec((n, k * f_out), lambda i: (0, 0)),
        ],
        out_specs=pl.BlockSpec((bm, k * f_out), lambda i: (i, 0)),
        out_shape=jax.ShapeDtypeStruct((n, k * f_out), jnp.float32),
        compiler_params=pltpu.CompilerParams(
            dimension_semantics=("parallel",)),
    )(adj, s)
    return out2d.reshape(n, k, f_out)
